# baseline (speedup 1.0000x reference)
import functools

import jax
import jax.numpy as jnp
from jax import lax
from jax.experimental import pallas as pl
from jax.experimental.pallas import tpu as pltpu
from jax.experimental.pallas import tpu_sc as plsc

NC = 2
NS = 16
NW = NC * NS
LANES = 16
NQ = 4
ROWL = 128


def _sc_aggregate(neigh4, features, B, deg):
  D = features.shape[1]
  assert B % NW == 0 and (B * deg) % ROWL == 0
  b_per_w = B // NW
  qrows = b_per_w // NQ
  rows_per_w = b_per_w * deg // ROWL

  mesh = plsc.VectorSubcoreMesh(core_axis_name="c", subcore_axis_name="s",
                                num_cores=NC, num_subcores=NS)

  @functools.partial(
      pl.kernel,
      mesh=mesh,
      compiler_params=pltpu.CompilerParams(needs_layout_passes=False,
                                          use_tc_tiling_on_sc=True),
      out_type=jax.ShapeDtypeStruct((B, D), jnp.float32),
      scratch_types=[
          pltpu.VMEM((rows_per_w, ROWL), jnp.int32),
          pltpu.VMEM((deg, b_per_w), jnp.int32),
          pltpu.VMEM((NQ, qrows, D), jnp.float32),
          pltpu.SemaphoreType.DMA,
          pltpu.SemaphoreType.DMA((NQ,)),
      ],
  )
  def agg_kernel(neigh4_hbm, feat_hbm, out_hbm,
                 nst_v, adjt_v, acc_v, sem0, qsems):
    wid = lax.axis_index("s") * NC + lax.axis_index("c")
    base = wid * b_per_w

    pltpu.sync_copy(neigh4_hbm.at[pl.ds(wid * rows_per_w, rows_per_w)], nst_v)

    lane = lax.iota(jnp.int32, LANES)

    def tr0_body(i, _):
      flat = (i * LANES + lane) * deg
      vals = plsc.load_gather(nst_v, [flat // ROWL, flat % ROWL])
      adjt_v[0, pl.ds(i * LANES, LANES)] = vals
      return 0

    lax.fori_loop(0, b_per_w // LANES, tr0_body, 0)

    def gadd(r, q, add):
      pltpu.async_copy(
          feat_hbm.at[adjt_v.at[r, pl.ds(q * qrows, qrows)]], acc_v.at[q],
          qsems.at[q], add=add)

    for q in range(NQ):
      gadd(0, q, False)

    def tr_body(i, _):
      flat0 = (i * LANES + lane) * deg
      for r in range(1, deg):
        flat = flat0 + r
        vals = plsc.load_gather(nst_v, [flat // ROWL, flat % ROWL])
        adjt_v[r, pl.ds(i * LANES, LANES)] = vals
      return 0

    lax.fori_loop(0, b_per_w // LANES, tr_body, 0)

    def r_body(r, _):
      for q in range(NQ):
        pltpu.make_async_copy(
            feat_hbm.at[adjt_v.at[0, pl.ds(q * qrows, qrows)]], acc_v.at[q],
            qsems.at[q]).wait()

        @pl.when(r < deg)
        def _():
          gadd(r, q, True)
      return 0

    lax.fori_loop(1, deg + 1, r_body, 0)

    for q in range(NQ):
      pltpu.sync_copy(acc_v.at[q], out_hbm.at[pl.ds(base + q * qrows, qrows)])

  return agg_kernel(neigh4, features)


def _tc_tail(agg, W, scale):
  B, D = agg.shape
  BLK = 2048
  grid = B // BLK

  def body(a_ref, w_ref, o_ref):
    a = a_ref[...] * scale
    h = jnp.dot(a, w_ref[...], preferred_element_type=jnp.float32)
    h = jnp.maximum(h, 0.0)
    norm = jnp.sqrt(jnp.sum(h * h, axis=1, keepdims=True))
    o_ref[...] = h / jnp.maximum(norm, 1e-12)

  return pl.pallas_call(
      body,
      grid=(grid,),
      in_specs=[
          pl.BlockSpec((BLK, D), lambda i: (i, 0)),
          pl.BlockSpec((D, D), lambda i: (0, 0)),
      ],
      out_specs=pl.BlockSpec((BLK, D), lambda i: (i, 0)),
      out_shape=jax.ShapeDtypeStruct((B, D), jnp.float32),
  )(agg, W)


@jax.jit
def kernel(nodes, features, adj, W):
  nodes = nodes.astype(jnp.int32)
  B = nodes.shape[0]
  deg = adj.shape[1]
  neigh = jnp.take(adj, nodes, axis=0)
  neigh4 = neigh.reshape(-1, ROWL)
  aggsum = _sc_aggregate(neigh4, features, B, deg)
  return _tc_tail(aggsum, W, 1.0 / deg)


# speedup vs baseline: 1.0568x; 1.0568x over previous
import functools

import jax
import jax.numpy as jnp
from jax import lax
from jax.experimental import pallas as pl
from jax.experimental.pallas import tpu as pltpu
from jax.experimental.pallas import tpu_sc as plsc

NC = 2
NS = 16
NW = NC * NS
LANES = 16
NQ = 8
ROWL = 128


def _sc_aggregate(neigh4, features, B, deg):
  D = features.shape[1]
  assert B % NW == 0 and (B * deg) % ROWL == 0
  b_per_w = B // NW
  qrows = b_per_w // NQ
  rows_per_w = b_per_w * deg // ROWL

  mesh = plsc.VectorSubcoreMesh(core_axis_name="c", subcore_axis_name="s",
                                num_cores=NC, num_subcores=NS)

  @functools.partial(
      pl.kernel,
      mesh=mesh,
      compiler_params=pltpu.CompilerParams(needs_layout_passes=False,
                                          use_tc_tiling_on_sc=True),
      out_type=jax.ShapeDtypeStruct((B, D), jnp.float32),
      scratch_types=[
          pltpu.VMEM((rows_per_w, ROWL), jnp.int32),
          pltpu.VMEM((deg, b_per_w), jnp.int32),
          pltpu.VMEM((NQ, qrows, D), jnp.float32),
          pltpu.SemaphoreType.DMA,
          pltpu.SemaphoreType.DMA((NQ,)),
      ],
  )
  def agg_kernel(neigh4_hbm, feat_hbm, out_hbm,
                 nst_v, adjt_v, acc_v, sem0, qsems):
    wid = lax.axis_index("s") * NC + lax.axis_index("c")
    base = wid * b_per_w

    pltpu.sync_copy(neigh4_hbm.at[pl.ds(wid * rows_per_w, rows_per_w)], nst_v)

    lane = lax.iota(jnp.int32, LANES)

    def tr0_body(i, _):
      flat = (i * LANES + lane) * deg
      vals = plsc.load_gather(nst_v, [flat // ROWL, flat % ROWL])
      adjt_v[0, pl.ds(i * LANES, LANES)] = vals
      return 0

    lax.fori_loop(0, b_per_w // LANES, tr0_body, 0)

    def gadd(r, q, add):
      pltpu.async_copy(
          feat_hbm.at[adjt_v.at[r, pl.ds(q * qrows, qrows)]], acc_v.at[q],
          qsems.at[q], add=add)

    for q in range(NQ):
      gadd(0, q, False)

    def tr_body(i, _):
      flat0 = (i * LANES + lane) * deg
      for r in range(1, deg):
        flat = flat0 + r
        vals = plsc.load_gather(nst_v, [flat // ROWL, flat % ROWL])
        adjt_v[r, pl.ds(i * LANES, LANES)] = vals
      return 0

    lax.fori_loop(0, b_per_w // LANES, tr_body, 0)

    def r_body(r, _):
      for q in range(NQ):
        pltpu.make_async_copy(
            feat_hbm.at[adjt_v.at[0, pl.ds(q * qrows, qrows)]], acc_v.at[q],
            qsems.at[q]).wait()

        @pl.when(r < deg)
        def _():
          gadd(r, q, True)
      return 0

    lax.fori_loop(1, deg + 1, r_body, 0)

    for q in range(NQ):
      pltpu.sync_copy(acc_v.at[q], out_hbm.at[pl.ds(base + q * qrows, qrows)])

  return agg_kernel(neigh4, features)


def _tc_tail(agg, W, scale):
  B, D = agg.shape
  BLK = 4096
  grid = B // BLK

  def body(a_ref, w_ref, o_ref):
    a = a_ref[...] * scale
    h = jnp.dot(a, w_ref[...], preferred_element_type=jnp.float32)
    h = jnp.maximum(h, 0.0)
    norm = jnp.sqrt(jnp.sum(h * h, axis=1, keepdims=True))
    o_ref[...] = h / jnp.maximum(norm, 1e-12)

  return pl.pallas_call(
      body,
      grid=(grid,),
      in_specs=[
          pl.BlockSpec((BLK, D), lambda i: (i, 0)),
          pl.BlockSpec((D, D), lambda i: (0, 0)),
      ],
      out_specs=pl.BlockSpec((BLK, D), lambda i: (i, 0)),
      out_shape=jax.ShapeDtypeStruct((B, D), jnp.float32),
  )(agg, W)


@jax.jit
def kernel(nodes, features, adj, W):
  nodes = nodes.astype(jnp.int32)
  B = nodes.shape[0]
  deg = adj.shape[1]
  neigh = jnp.take(adj, nodes, axis=0)
  neigh4 = neigh.reshape(-1, ROWL)
  aggsum = _sc_aggregate(neigh4, features, B, deg)
  return _tc_tail(aggsum, W, 1.0 / deg)


# speedup vs baseline: 1.0682x; 1.0108x over previous
import functools

import jax
import jax.numpy as jnp
from jax import lax
from jax.experimental import pallas as pl
from jax.experimental.pallas import tpu as pltpu
from jax.experimental.pallas import tpu_sc as plsc

NC = 2
NS = 16
NW = NC * NS
LANES = 16
NQ = 8
ROWL = 128


def _sc_aggregate(neigh4, features, B, deg):
  D = features.shape[1]
  assert B % NW == 0 and (B * deg) % ROWL == 0
  b_per_w = B // NW
  qrows = b_per_w // NQ
  rows_per_w = b_per_w * deg // ROWL

  mesh = plsc.VectorSubcoreMesh(core_axis_name="c", subcore_axis_name="s",
                                num_cores=NC, num_subcores=NS)

  @functools.partial(
      pl.kernel,
      mesh=mesh,
      compiler_params=pltpu.CompilerParams(needs_layout_passes=False,
                                          use_tc_tiling_on_sc=True),
      out_type=jax.ShapeDtypeStruct((B, D), jnp.float32),
      scratch_types=[
          pltpu.VMEM((rows_per_w, ROWL), jnp.int32),
          pltpu.VMEM((deg, b_per_w), jnp.int32),
          pltpu.VMEM((NQ, qrows, D), jnp.float32),
          pltpu.SemaphoreType.DMA,
          pltpu.SemaphoreType.DMA((NQ,)),
      ],
  )
  def agg_kernel(neigh4_hbm, feat_hbm, out_hbm,
                 nst_v, adjt_v, acc_v, sem0, qsems):
    wid = lax.axis_index("s") * NC + lax.axis_index("c")
    base = wid * b_per_w

    pltpu.sync_copy(neigh4_hbm.at[pl.ds(wid * rows_per_w, rows_per_w)], nst_v)

    lane = lax.iota(jnp.int32, LANES)

    def tr0_body(i, _):
      flat = (i * LANES + lane) * deg
      vals = plsc.load_gather(nst_v, [flat // ROWL, flat % ROWL])
      adjt_v[0, pl.ds(i * LANES, LANES)] = vals
      return 0

    lax.fori_loop(0, b_per_w // LANES, tr0_body, 0)

    def gadd(r, q, add):
      pltpu.async_copy(
          feat_hbm.at[adjt_v.at[r, pl.ds(q * qrows, qrows)]], acc_v.at[q],
          qsems.at[q], add=add)

    for q in range(NQ):
      gadd(0, q, False)

    def tr_body(i, _):
      flat0 = (i * LANES + lane) * deg
      for r in range(1, deg):
        flat = flat0 + r
        vals = plsc.load_gather(nst_v, [flat // ROWL, flat % ROWL])
        adjt_v[r, pl.ds(i * LANES, LANES)] = vals
      return 0

    lax.fori_loop(0, b_per_w // LANES, tr_body, 0)

    def r_body(r, _):
      for q in range(NQ):
        pltpu.make_async_copy(
            feat_hbm.at[adjt_v.at[0, pl.ds(q * qrows, qrows)]], acc_v.at[q],
            qsems.at[q]).wait()

        @pl.when(r < deg)
        def _():
          gadd(r, q, True)
      return 0

    lax.fori_loop(1, deg + 1, r_body, 0)

    for q in range(NQ):
      pltpu.sync_copy(acc_v.at[q], out_hbm.at[pl.ds(base + q * qrows, qrows)])

  return agg_kernel(neigh4, features)


def _tc_tail(agg, W, scale):
  B, D = agg.shape
  BLK = 8192
  grid = B // BLK

  def body(a_ref, w_ref, o_ref):
    a = a_ref[...] * scale
    h = jnp.dot(a, w_ref[...], preferred_element_type=jnp.float32)
    h = jnp.maximum(h, 0.0)
    norm = jnp.sqrt(jnp.sum(h * h, axis=1, keepdims=True))
    o_ref[...] = h / jnp.maximum(norm, 1e-12)

  return pl.pallas_call(
      body,
      grid=(grid,),
      in_specs=[
          pl.BlockSpec((BLK, D), lambda i: (i, 0)),
          pl.BlockSpec((D, D), lambda i: (0, 0)),
      ],
      out_specs=pl.BlockSpec((BLK, D), lambda i: (i, 0)),
      out_shape=jax.ShapeDtypeStruct((B, D), jnp.float32),
  )(agg, W)


@jax.jit
def kernel(nodes, features, adj, W):
  nodes = nodes.astype(jnp.int32)
  B = nodes.shape[0]
  deg = adj.shape[1]
  neigh = jnp.take(adj, nodes, axis=0)
  neigh4 = neigh.reshape(-1, ROWL)
  aggsum = _sc_aggregate(neigh4, features, B, deg)
  return _tc_tail(aggsum, W, 1.0 / deg)
